# baseline (speedup 1.0000x reference)
import functools

import jax
import jax.numpy as jnp
from jax import lax
from jax.experimental import pallas as pl
from jax.experimental.pallas import tpu as pltpu
from jax.experimental.pallas import tpu_sc as plsc

_NUM_CORES = 2
_NUM_SUBCORES = 16
_NW = _NUM_CORES * _NUM_SUBCORES

_SUB = 16
_LANE = 128


def _pair_table(W):
  vocab, dim = W.shape
  blk = 1600
  grid = vocab // blk

  def body(w_ref, o_ref):
    o_ref[:, :dim] = w_ref[0::2, :]
    o_ref[:, dim:] = w_ref[1::2, :]

  return pl.pallas_call(
      body,
      grid=(grid,),
      in_specs=[pl.BlockSpec((blk, dim), lambda i: (i, 0))],
      out_specs=pl.BlockSpec((blk // 2, 2 * dim), lambda i: (i, 0)),
      out_shape=jax.ShapeDtypeStruct((vocab // 2, 2 * dim), jnp.float32),
  )(W)


def _split_pairs(out128, batch, seq, dim):
  nb = 8
  grid = batch // nb
  rows = nb * seq // 2

  def body(i_ref, o_ref):
    for b in range(nb):
      blkrows = i_ref[pl.ds(b * seq // 2, seq // 2), :]
      o_ref[b, 0::2, :] = blkrows[:, :dim]
      o_ref[b, 1::2, :] = blkrows[:, dim:]

  return pl.pallas_call(
      body,
      grid=(grid,),
      in_specs=[pl.BlockSpec((rows, 2 * dim), lambda i: (i, 0))],
      out_specs=pl.BlockSpec((nb, seq, dim), lambda i: (i, 0, 0)),
      out_shape=jax.ShapeDtypeStruct((batch, seq, dim), jnp.float32),
  )(out128)


def _build(batch, seq, dim):
  b_per_w = batch // _NW
  n_per_w = b_per_w * seq
  n_grp = (seq + _SUB - 1) // _SUB
  pad = n_grp * _SUB
  g_rows = 224
  p_per_b = seq // 2
  mesh = plsc.VectorSubcoreMesh(core_axis_name="c", subcore_axis_name="s")

  @functools.partial(
      pl.kernel,
      out_type=jax.ShapeDtypeStruct((batch * p_per_b, 2 * dim), jnp.float32),
      mesh=mesh,
      scratch_types=[
          pltpu.VMEM((n_per_w + 2 * _SUB,), jnp.int32),
          pltpu.VMEM((2, 2, _LANE), jnp.int32),
          pltpu.VMEM((2, g_rows, _LANE), jnp.float32),
          pltpu.VMEM((pad // 2, 2 * dim), jnp.float32),
          pltpu.SemaphoreType.DMA,
          pltpu.SemaphoreType.DMA((2,)),
          pltpu.SemaphoreType.DMA,
      ],
      compiler_params=pltpu.CompilerParams(use_tc_tiling_on_sc=False),
  )
  def lookup(ids_hbm, pairs_hbm, out_hbm, idx_v, q_v, g_v, o_v,
             isem, gsem, osem):
    wid = lax.axis_index("s") * _NUM_CORES + lax.axis_index("c")
    base = pl.multiple_of(wid * n_per_w, _LANE)
    b0 = wid * b_per_w

    zeros16 = jnp.zeros((_SUB,), jnp.int32)
    idx_v[pl.ds(n_per_w, _SUB)] = zeros16
    idx_v[pl.ds(n_per_w + _SUB, _SUB)] = zeros16
    pltpu.async_copy(ids_hbm.at[pl.ds(base, n_per_w)],
                     idx_v.at[pl.ds(0, n_per_w)], isem)
    pltpu.make_async_copy(ids_hbm.at[pl.ds(base, n_per_w)],
                          idx_v.at[pl.ds(0, n_per_w)], isem).wait()

    def compute_q(k, s):
      for c in range((_LANE + 96) // _SUB):
        ids16 = idx_v[pl.ds(k * seq + c * _SUB, _SUB)]
        q_v[s, c // (_LANE // _SUB), pl.ds((c % (_LANE // _SUB)) * _SUB,
                                           _SUB)] = (
            lax.shift_right_logical(ids16, 1))

    def fire_gather(k, s):
      compute_q(k, s)
      pltpu.async_copy(pairs_hbm.at[q_v.at[s].at[0]],
                       g_v.at[s].at[pl.ds(0, _LANE)], gsem.at[s])
      pltpu.async_copy(pairs_hbm.at[q_v.at[s].at[1].at[pl.ds(0, 96)]],
                       g_v.at[s].at[pl.ds(_LANE, 96)], gsem.at[s])

    def wait_gather(s):
      pltpu.make_async_copy(
          pairs_hbm.at[q_v.at[s].at[0]],
          g_v.at[s].at[pl.ds(0, _LANE)], gsem.at[s]).wait()
      pltpu.make_async_copy(
          pairs_hbm.at[q_v.at[s].at[1].at[pl.ds(0, 96)]],
          g_v.at[s].at[pl.ds(_LANE, 96)], gsem.at[s]).wait()

    def writeout(k):
      return pltpu.make_async_copy(
          o_v.at[pl.ds(0, p_per_b)],
          out_hbm.at[pl.ds((b0 + k) * p_per_b, p_per_b)], osem)

    def extract(k, s):
      def group(g, carry):
        ids16 = idx_v[pl.ds(k * seq + g * _SUB, _SUB)]
        par16 = ids16 & 1
        parf16 = par16.astype(jnp.float32)
        for l in range(_SUB):
          spl = jnp.take(parf16, jnp.full((_SUB,), l, jnp.int32))
          cospl = 1.0 - spl
          row = g * _SUB + l
          prow = g * (_SUB // 2) + l // 2
          cbase = (l % 2) * dim
          for c in range(dim // _SUB):
            lo = g_v[s, row, pl.ds(c * _SUB, _SUB)]
            hi = g_v[s, row, pl.ds(dim + c * _SUB, _SUB)]
            o_v[prow, pl.ds(cbase + c * _SUB, _SUB)] = (
                lo * cospl + hi * spl)
        return carry

      lax.fori_loop(0, n_grp, group, 0)

    fire_gather(0, 0)

    def body(t, carry):
      for j in range(2):
        k = t * 2 + j
        s = j
        @pl.when(k + 1 < b_per_w)
        def _():
          fire_gather(k + 1, 1 - s)
        wait_gather(s)
        @pl.when(k >= 1)
        def _():
          writeout(k - 1).wait()
        extract(k, s)
        writeout(k).start()
      return carry

    lax.fori_loop(0, b_per_w // 2, body, 0)

    writeout(b_per_w - 1).wait()

  return lookup


def kernel(token_ids, W):
  b, s = token_ids.shape
  vocab, dim = W.shape
  ids = token_ids.reshape(b * s).astype(jnp.int32)
  pairs = _pair_table(W)
  out128 = _build(b, s, dim)(ids, pairs)
  return _split_pairs(out128, b, s, dim)


# speedup vs baseline: 1.8780x; 1.8780x over previous
import functools

import jax
import jax.numpy as jnp
from jax import lax
from jax.experimental import pallas as pl
from jax.experimental.pallas import tpu as pltpu
from jax.experimental.pallas import tpu_sc as plsc

_NUM_CORES = 2
_NUM_SUBCORES = 16
_NW = _NUM_CORES * _NUM_SUBCORES

_SUB = 16
_LANE = 128


def _build(batch, seq, dim):
  b_per_w = batch // _NW
  n_per_w = b_per_w * seq
  n_grp = (seq + _SUB - 1) // _SUB
  pad = n_grp * _SUB
  g_rows = 224
  mesh = plsc.VectorSubcoreMesh(core_axis_name="c", subcore_axis_name="s")

  @functools.partial(
      pl.kernel,
      out_type=jax.ShapeDtypeStruct((batch, seq, dim), jnp.float32),
      mesh=mesh,
      scratch_types=[
          pltpu.VMEM((n_per_w + 2 * _SUB,), jnp.int32),
          pltpu.VMEM((2, 2, _LANE), jnp.int32),
          pltpu.VMEM((2, g_rows, _LANE), jnp.float32),
          pltpu.VMEM((pad, dim), jnp.float32),
          pltpu.SemaphoreType.DMA,
          pltpu.SemaphoreType.DMA((2,)),
          pltpu.SemaphoreType.DMA,
      ],
      compiler_params=pltpu.CompilerParams(use_tc_tiling_on_sc=True),
  )
  def lookup(ids_hbm, pairs_hbm, out_hbm, idx_v, q_v, g_v, o_v,
             isem, gsem, osem):
    wid = lax.axis_index("s") * _NUM_CORES + lax.axis_index("c")
    base = pl.multiple_of(wid * n_per_w, _LANE)
    b0 = wid * b_per_w

    zeros16 = jnp.zeros((_SUB,), jnp.int32)
    idx_v[pl.ds(n_per_w, _SUB)] = zeros16
    idx_v[pl.ds(n_per_w + _SUB, _SUB)] = zeros16
    pltpu.async_copy(ids_hbm.at[pl.ds(base, n_per_w)],
                     idx_v.at[pl.ds(0, n_per_w)], isem)
    pltpu.make_async_copy(ids_hbm.at[pl.ds(base, n_per_w)],
                          idx_v.at[pl.ds(0, n_per_w)], isem).wait()

    def compute_q(k, s):
      for c in range((_LANE + 96) // _SUB):
        ids16 = idx_v[pl.ds(k * seq + c * _SUB, _SUB)]
        q_v[s, c // (_LANE // _SUB), pl.ds((c % (_LANE // _SUB)) * _SUB,
                                           _SUB)] = (
            lax.shift_right_logical(ids16, 1))

    def fire_gather(k, s):
      compute_q(k, s)
      pltpu.async_copy(pairs_hbm.at[q_v.at[s].at[0]],
                       g_v.at[s].at[pl.ds(0, _LANE)], gsem.at[s])
      pltpu.async_copy(pairs_hbm.at[q_v.at[s].at[1].at[pl.ds(0, 96)]],
                       g_v.at[s].at[pl.ds(_LANE, 96)], gsem.at[s])

    def wait_gather(s):
      pltpu.make_async_copy(
          pairs_hbm.at[q_v.at[s].at[0]],
          g_v.at[s].at[pl.ds(0, _LANE)], gsem.at[s]).wait()
      pltpu.make_async_copy(
          pairs_hbm.at[q_v.at[s].at[1].at[pl.ds(0, 96)]],
          g_v.at[s].at[pl.ds(_LANE, 96)], gsem.at[s]).wait()

    def writeout(k):
      return pltpu.make_async_copy(
          o_v.at[pl.ds(0, seq)], out_hbm.at[b0 + k], osem)

    def extract(k, s):
      def group(g, carry):
        ids16 = idx_v[pl.ds(k * seq + g * _SUB, _SUB)]
        par16 = ids16 & 1
        parf16 = par16.astype(jnp.float32)
        for l in range(_SUB):
          spl = jnp.take(parf16, jnp.full((_SUB,), l, jnp.int32))
          cospl = 1.0 - spl
          row = g * _SUB + l
          for c in range(dim // _SUB):
            lo = g_v[s, row, pl.ds(c * _SUB, _SUB)]
            hi = g_v[s, row, pl.ds(dim + c * _SUB, _SUB)]
            o_v[row, pl.ds(c * _SUB, _SUB)] = lo * cospl + hi * spl
        return carry

      lax.fori_loop(0, n_grp, group, 0)

    fire_gather(0, 0)

    def body(t, carry):
      for j in range(2):
        k = t * 2 + j
        s = j
        @pl.when(k + 1 < b_per_w)
        def _():
          fire_gather(k + 1, 1 - s)
        wait_gather(s)
        @pl.when(k >= 1)
        def _():
          writeout(k - 1).wait()
        extract(k, s)
        writeout(k).start()
      return carry

    lax.fori_loop(0, b_per_w // 2, body, 0)

    writeout(b_per_w - 1).wait()

  return lookup


def kernel(token_ids, W):
  b, s = token_ids.shape
  vocab, dim = W.shape
  ids = token_ids.reshape(b * s).astype(jnp.int32)
  pairs = W.reshape(vocab // 2, 2 * dim)
  return _build(b, s, dim)(ids, pairs)
